# Initial kernel scaffold; baseline (speedup 1.0000x reference)
import jax
import jax.numpy as jnp
from jax.experimental import pallas as pl


def kernel(x, in_proj_w, in_proj_b, out_proj_w, out_proj_b, content_w, content_b, k_input):
    raise NotImplementedError("write your pallas kernel here")


# speedup vs baseline: 1.5966x; 1.5966x over previous
import jax
import jax.numpy as jnp
import numpy as np
from jax.experimental import pallas as pl
from jax.experimental.pallas import tpu as pltpu

D_MODEL = 1024
N_HEADS = 16
DH = D_MODEL // N_HEADS
N_INPUT = 64
K_TOP = 8
B = 2
S = 2048
BM = 256
BQ = 512
NBLK = (B * S) // BM
BLK_PER_BATCH = S // BM


def _qkv_kernel(x_ref, w_ref, b_ref, out_ref):
    x = x_ref[...]
    w = w_ref[...]
    acc = jax.lax.dot_general(x, w, (((1,), (1,)), ((), ())),
                              preferred_element_type=jnp.float32)
    acc = acc + b_ref[...]
    out_ref[0, 0] = acc.reshape(BM, N_HEADS, DH)


def _attn_kernel(q_ref, k_ref, v_ref, o_ref):
    q = q_ref[0]
    k = k_ref[0]
    v = v_ref[0]
    s = jax.lax.dot_general(q, k, (((1,), (1,)), ((), ())),
                            preferred_element_type=jnp.float32)
    s = s * jnp.float32(1.0 / np.sqrt(DH))
    m = jnp.max(s, axis=1, keepdims=True)
    p = jnp.exp(s - m)
    l = jnp.sum(p, axis=1, keepdims=True)
    p = p / l
    o_ref[0] = jnp.dot(p, v, preferred_element_type=jnp.float32)


def _out_kernel(ctx_ref, wo_ref, bo_ref, wc_ref, bc_ref,
                att_ref, idx_ref, rw_ref, ssum, smax):
    i = pl.program_id(0)
    ctxb = ctx_ref[...]
    att = jax.lax.dot_general(ctxb, wo_ref[...], (((1,), (1,)), ((), ())),
                              preferred_element_type=jnp.float32)
    att = att + bo_ref[...]
    att_ref[...] = att
    aff = jax.lax.dot_general(att, wc_ref[...], (((1,), (1,)), ((), ())),
                              preferred_element_type=jnp.float32)
    aff = aff + bc_ref[...]
    psum = jnp.sum(aff, axis=0, keepdims=True)
    pmax = jnp.max(aff, axis=0, keepdims=True)
    b = i // BLK_PER_BATCH
    rows = jax.lax.broadcasted_iota(jnp.int32, (B, 1), 0)
    mask = rows == b

    @pl.when(i == 0)
    def _():
        ssum[...] = jnp.zeros((B, N_INPUT), jnp.float32)
        smax[...] = jnp.full((B, N_INPUT), -jnp.inf, jnp.float32)

    ssum[...] = ssum[...] + jnp.where(mask, psum, 0.0)
    smax[...] = jnp.maximum(smax[...], jnp.where(mask, pmax, -jnp.inf))

    @pl.when(i == NBLK - 1)
    def _():
        ss = ssum[...]
        sm = smax[...]
        inv = jnp.float32(1.0 / (S + 1e-8))
        final = 0.5 * (ss * inv) + 0.3 * sm + 0.2 * (ss * jnp.float32(1.0 / S))
        fm = jnp.max(final, axis=1, keepdims=True)
        pe = jnp.exp(final - fm)
        probs = pe / jnp.sum(pe, axis=1, keepdims=True)
        cols = jax.lax.broadcasted_iota(jnp.int32, (B, N_INPUT), 1)
        run = final
        oh = jnp.zeros((B, N_INPUT), jnp.float32)
        for j in range(K_TOP):
            cm = jnp.max(run, axis=1, keepdims=True)
            idx = jnp.min(jnp.where(run == cm, cols, N_INPUT),
                          axis=1, keepdims=True)
            idx_ref[:, j:j + 1] = idx
            sel = cols == idx
            oh = jnp.where(sel, 1.0, oh)
            run = jnp.where(sel, -jnp.inf, run)
        rw_ref[...] = (oh - probs) + probs


def kernel(x, in_proj_w, in_proj_b, out_proj_w, out_proj_b,
           content_w, content_b, k_input):
    x2d = x.reshape(B * S, D_MODEL)

    qkv3 = pl.pallas_call(
        _qkv_kernel,
        grid=(3, NBLK),
        in_specs=[
            pl.BlockSpec((BM, D_MODEL), lambda j, i: (i, 0)),
            pl.BlockSpec((D_MODEL, D_MODEL), lambda j, i: (j, 0)),
            pl.BlockSpec((1, D_MODEL), lambda j, i: (0, j)),
        ],
        out_specs=pl.BlockSpec(
            (1, 1, BM, N_HEADS, DH),
            lambda j, i: (j, i // BLK_PER_BATCH, i % BLK_PER_BATCH, 0, 0)),
        out_shape=jax.ShapeDtypeStruct((3, B, S, N_HEADS, DH), jnp.float32),
    )(x2d, in_proj_w, in_proj_b.reshape(1, 3 * D_MODEL))

    q3 = qkv3[0].transpose(0, 2, 1, 3).reshape(B * N_HEADS, S, DH)
    k3 = qkv3[1].transpose(0, 2, 1, 3).reshape(B * N_HEADS, S, DH)
    v3 = qkv3[2].transpose(0, 2, 1, 3).reshape(B * N_HEADS, S, DH)

    ctx_h = pl.pallas_call(
        _attn_kernel,
        grid=(B * N_HEADS, S // BQ),
        in_specs=[
            pl.BlockSpec((1, BQ, DH), lambda bh, qi: (bh, qi, 0)),
            pl.BlockSpec((1, S, DH), lambda bh, qi: (bh, 0, 0)),
            pl.BlockSpec((1, S, DH), lambda bh, qi: (bh, 0, 0)),
        ],
        out_specs=pl.BlockSpec((1, BQ, DH), lambda bh, qi: (bh, qi, 0)),
        out_shape=jax.ShapeDtypeStruct((B * N_HEADS, S, DH), jnp.float32),
    )(q3, k3, v3)

    ctx2d = (ctx_h.reshape(B, N_HEADS, S, DH)
             .transpose(0, 2, 1, 3).reshape(B * S, D_MODEL))

    att2d, input_idx, routing_weights = pl.pallas_call(
        _out_kernel,
        grid=(NBLK,),
        in_specs=[
            pl.BlockSpec((BM, D_MODEL), lambda i: (i, 0)),
            pl.BlockSpec((D_MODEL, D_MODEL), lambda i: (0, 0)),
            pl.BlockSpec((1, D_MODEL), lambda i: (0, 0)),
            pl.BlockSpec((N_INPUT, D_MODEL), lambda i: (0, 0)),
            pl.BlockSpec((1, N_INPUT), lambda i: (0, 0)),
        ],
        out_specs=[
            pl.BlockSpec((BM, D_MODEL), lambda i: (i, 0)),
            pl.BlockSpec((B, K_TOP), lambda i: (0, 0)),
            pl.BlockSpec((B, N_INPUT), lambda i: (0, 0)),
        ],
        out_shape=[
            jax.ShapeDtypeStruct((B * S, D_MODEL), jnp.float32),
            jax.ShapeDtypeStruct((B, K_TOP), jnp.int32),
            jax.ShapeDtypeStruct((B, N_INPUT), jnp.float32),
        ],
        scratch_shapes=[
            pltpu.VMEM((B, N_INPUT), jnp.float32),
            pltpu.VMEM((B, N_INPUT), jnp.float32),
        ],
    )(ctx2d, out_proj_w, out_proj_b.reshape(1, D_MODEL),
      content_w, content_b.reshape(1, N_INPUT))

    attended = att2d.reshape(B, S, D_MODEL)
    return input_idx, routing_weights, attended
